# baseline (speedup 1.0000x reference)
import functools

import jax
import jax.numpy as jnp
from jax import lax
from jax.experimental import pallas as pl
from jax.experimental.pallas import tpu as pltpu
from jax.experimental.pallas import tpu_sc as plsc

_EPS = 1e-5


def _argmin_loss_body(x_ref, wt_ref, idx_ref, loss_ref):
    x = x_ref[0]
    wt = wt_ref[...]
    T, C = x.shape
    K = wt.shape[1]

    s = lax.dot_general(x, wt, (((1,), (0,)), ((), ())),
                        preferred_element_type=jnp.float32)
    xs = jnp.sum(x * x, axis=1, keepdims=True)
    w2 = jnp.sum(wt * wt, axis=0, keepdims=True)
    d2 = (xs + w2) - 2.0 * s

    dmin = jnp.min(d2, axis=1, keepdims=True)
    eqm = d2 == dmin
    kiota = lax.broadcasted_iota(jnp.int32, (T, K), 1).astype(jnp.float32)
    idx = jnp.min(jnp.where(eqm, kiota, float(K)), axis=1,
                  keepdims=True)
    idx_ref[0] = idx.astype(jnp.int32)

    ef = eqm.astype(jnp.float32)
    w2_win = lax.dot_general(ef, w2, (((1,), (1,)), ((), ())),
                             preferred_element_type=jnp.float32)
    s_win = 0.5 * ((xs + w2_win) - dmin)
    xn = jnp.maximum(jnp.sqrt(xs), _EPS)
    wn = jnp.maximum(jnp.sqrt(w2_win), _EPS)
    row = (w2_win / (wn * wn) + xs / (xn * xn)
           - 2.0 * s_win / (wn * xn))
    loss_ref[pl.program_id(0)] = jnp.sum(row) / (T * C)


def _argmin_and_loss(x, W):
    B, T, C = x.shape
    K = W.shape[0]
    idx3, loss = pl.pallas_call(
        _argmin_loss_body,
        grid=(B,),
        in_specs=[
            pl.BlockSpec((1, T, C), lambda b: (b, 0, 0)),
            pl.BlockSpec((C, K), lambda b: (0, 0)),
        ],
        out_specs=[
            pl.BlockSpec((1, T, 1), lambda b: (b, 0, 0)),
            pl.BlockSpec(memory_space=pltpu.SMEM),
        ],
        out_shape=[
            jax.ShapeDtypeStruct((B, T, 1), jnp.int32),
            jax.ShapeDtypeStruct((B,), jnp.float32),
        ],
    )(x, W.T)
    return idx3.reshape(B, T), loss


@functools.cache
def _make_sc_gather(V, D, B):
    info = plsc.get_sparse_core_info()
    NC, NS = info.num_cores, info.num_subcores
    NW = NC * NS
    assert B % (8 * NW) == 0
    b_per_w = B // NW
    mesh = plsc.VectorSubcoreMesh(core_axis_name="c", subcore_axis_name="s")

    @functools.partial(
        pl.kernel, mesh=mesh,
        out_type=jax.ShapeDtypeStruct((B, D), jnp.float32),
        scratch_types=[
            pltpu.VMEM((b_per_w,), jnp.int32),
            pltpu.VMEM((b_per_w, D), jnp.float32),
            pltpu.SemaphoreType.DMA,
        ],
    )
    def gather(table_hbm, idx_hbm, out_hbm, idx_v, rows_v, sem):
        wid = lax.axis_index("s") * NC + lax.axis_index("c")
        base = wid * b_per_w
        pltpu.sync_copy(idx_hbm.at[pl.ds(base, b_per_w)], idx_v)
        pltpu.async_copy(table_hbm.at[idx_v], rows_v, sem).wait()
        pltpu.sync_copy(rows_v, out_hbm.at[pl.ds(base, b_per_w)])

    return gather


def kernel(x, W):
    B, T, C = x.shape
    K = W.shape[0]
    H = B // 2
    gather = _make_sc_gather(K, C, H * T)
    idx_a, loss_a = _argmin_and_loss(x[:H], W)
    quant_a = gather(W, idx_a.reshape(-1))
    idx_b, loss_b = _argmin_and_loss(x[H:], W)
    quant_b = gather(W, idx_b.reshape(-1))
    quant = jnp.concatenate([quant_a, quant_b]).reshape(B, T, C)
    idx = jnp.concatenate([idx_a, idx_b])
    loss = jnp.concatenate([loss_a, loss_b])
    return quant, loss, loss, idx


# speedup vs baseline: 1.1372x; 1.1372x over previous
import functools

import jax
import jax.numpy as jnp
from jax import lax
from jax.experimental import pallas as pl
from jax.experimental.pallas import tpu as pltpu
from jax.experimental.pallas import tpu_sc as plsc

_EPS = 1e-5


def _argmin_loss_body(x_ref, wt_ref, idx_ref, loss_ref):
    x = x_ref[0]
    wt = wt_ref[...]
    T, C = x.shape
    K = wt.shape[1]

    s = lax.dot_general(x, wt, (((1,), (0,)), ((), ())),
                        preferred_element_type=jnp.float32)
    xs = jnp.sum(x * x, axis=1, keepdims=True)
    w2 = jnp.sum(wt * wt, axis=0, keepdims=True)
    d2 = (xs + w2) - 2.0 * s

    dmin = jnp.min(d2, axis=1, keepdims=True)
    eqm = d2 == dmin
    kiota = lax.broadcasted_iota(jnp.int32, (T, K), 1).astype(jnp.float32)
    idx = jnp.min(jnp.where(eqm, kiota, float(K)), axis=1,
                  keepdims=True)
    idx_ref[0] = idx.astype(jnp.int32)

    ef = eqm.astype(jnp.float32)
    w2_win = lax.dot_general(ef, w2, (((1,), (1,)), ((), ())),
                             preferred_element_type=jnp.float32)
    s_win = 0.5 * ((xs + w2_win) - dmin)
    xn = jnp.maximum(jnp.sqrt(xs), _EPS)
    wn = jnp.maximum(jnp.sqrt(w2_win), _EPS)
    row = (w2_win / (wn * wn) + xs / (xn * xn)
           - 2.0 * s_win / (wn * xn))
    loss_ref[pl.program_id(0)] = jnp.sum(row) / (T * C)


def _argmin_and_loss(x, Wt, off, nb):
    B, T, C = x.shape
    K = Wt.shape[1]
    idx3, loss = pl.pallas_call(
        _argmin_loss_body,
        grid=(nb,),
        in_specs=[
            pl.BlockSpec((1, T, C), lambda b: (b + off, 0, 0)),
            pl.BlockSpec((C, K), lambda b: (0, 0)),
        ],
        out_specs=[
            pl.BlockSpec((1, T, 1), lambda b: (b, 0, 0)),
            pl.BlockSpec(memory_space=pltpu.SMEM),
        ],
        out_shape=[
            jax.ShapeDtypeStruct((nb, T, 1), jnp.int32),
            jax.ShapeDtypeStruct((nb,), jnp.float32),
        ],
    )(x, Wt)
    return idx3.reshape(nb, T), loss


@functools.cache
def _make_sc_gather(V, D, B, OUT_ROWS):
    info = plsc.get_sparse_core_info()
    NC, NS = info.num_cores, info.num_subcores
    NW = NC * NS
    assert B % (8 * NW) == 0
    b_per_w = B // NW
    mesh = plsc.VectorSubcoreMesh(core_axis_name="c", subcore_axis_name="s")

    @functools.partial(
        pl.kernel, mesh=mesh,
        out_type=jax.ShapeDtypeStruct((OUT_ROWS, D), jnp.float32),
        scratch_types=[
            pltpu.VMEM((b_per_w,), jnp.int32),
            pltpu.VMEM((b_per_w, D), jnp.float32),
            pltpu.SemaphoreType.DMA,
        ],
    )
    def gather(table_hbm, idx_hbm, out_hbm, idx_v, rows_v, sem):
        wid = lax.axis_index("s") * NC + lax.axis_index("c")
        base = wid * b_per_w
        pltpu.sync_copy(idx_hbm.at[pl.ds(base, b_per_w)], idx_v)
        pltpu.async_copy(table_hbm.at[idx_v], rows_v, sem).wait()
        pltpu.sync_copy(rows_v, out_hbm.at[pl.ds(base, b_per_w)])

    return gather


def kernel(x, W):
    B, T, C = x.shape
    K = W.shape[0]
    H = B // 2
    Wt = W.T
    idx_a, loss_a = _argmin_and_loss(x, Wt, 0, H)
    quant_a = _make_sc_gather(K, C, H * T, B * T)(W, idx_a.reshape(-1))
    idx_b, loss_b = _argmin_and_loss(x, Wt, H, B - H)
    quant_b = _make_sc_gather(K, C, H * T, H * T)(W, idx_b.reshape(-1))
    quant = lax.dynamic_update_slice(quant_a, quant_b, (H * T, 0))
    quant = quant.reshape(B, T, C)
    idx = jnp.concatenate([idx_a, idx_b])
    loss = jnp.concatenate([loss_a, loss_b])
    return quant, loss, loss, idx


# speedup vs baseline: 1.2154x; 1.0688x over previous
import functools

import jax
import jax.numpy as jnp
from jax import lax
from jax.experimental import pallas as pl
from jax.experimental.pallas import tpu as pltpu
from jax.experimental.pallas import tpu_sc as plsc

_EPS = 1e-5
_BPP = 2


def _argmin_loss_body(x_ref, wt_ref, idx_ref, loss_ref):
    nb, T, C = x_ref.shape
    x = x_ref[...].reshape(nb * T, C)
    wt = wt_ref[...]
    K = wt.shape[1]
    R = nb * T

    s = lax.dot_general(x, wt, (((1,), (0,)), ((), ())),
                        preferred_element_type=jnp.float32)
    xs = jnp.sum(x * x, axis=1, keepdims=True)
    w2 = jnp.sum(wt * wt, axis=0, keepdims=True)
    d2 = (xs + w2) - 2.0 * s

    dmin = jnp.min(d2, axis=1, keepdims=True)
    eqm = d2 == dmin
    kiota = lax.broadcasted_iota(jnp.int32, (R, K), 1).astype(jnp.float32)
    idx = jnp.min(jnp.where(eqm, kiota, float(K)), axis=1)
    p = pl.program_id(0)
    idx_ref[pl.ds(p * R, R)] = idx.astype(jnp.int32)

    ef = eqm.astype(jnp.float32)
    w2_win = lax.dot_general(ef, w2, (((1,), (1,)), ((), ())),
                             preferred_element_type=jnp.float32)
    s_win = 0.5 * ((xs + w2_win) - dmin)
    xn = jnp.maximum(jnp.sqrt(xs), _EPS)
    wn = jnp.maximum(jnp.sqrt(w2_win), _EPS)
    row = (w2_win / (wn * wn) + xs / (xn * xn)
           - 2.0 * s_win / (wn * xn))
    row2 = row.reshape(nb, T)
    for j in range(nb):
        loss_ref[p * nb + j] = jnp.sum(row2[j]) / (T * C)


def _argmin_and_loss(x, Wt, off, nb):
    B, T, C = x.shape
    K = Wt.shape[1]
    grid = nb // _BPP
    boff = off // _BPP
    assert off % _BPP == 0
    idxf, loss = pl.pallas_call(
        _argmin_loss_body,
        grid=(grid,),
        in_specs=[
            pl.BlockSpec((_BPP, T, C), lambda b, boff=boff: (b + boff, 0, 0)),
            pl.BlockSpec((C, K), lambda b: (0, 0)),
        ],
        out_specs=[
            pl.BlockSpec((nb * T,), lambda b: (0,)),
            pl.BlockSpec(memory_space=pltpu.SMEM),
        ],
        out_shape=[
            jax.ShapeDtypeStruct((nb * T,), jnp.int32),
            jax.ShapeDtypeStruct((nb,), jnp.float32),
        ],
    )(x, Wt)
    return idxf, loss


@functools.cache
def _make_sc_gather(V, D, B, OUT_ROWS):
    info = plsc.get_sparse_core_info()
    NC, NS = info.num_cores, info.num_subcores
    NW = NC * NS
    assert B % (8 * NW) == 0
    b_per_w = B // NW
    NCH = 3
    CH = b_per_w // NCH
    assert CH % 8 == 0
    mesh = plsc.VectorSubcoreMesh(core_axis_name="c", subcore_axis_name="s")

    @functools.partial(
        pl.kernel, mesh=mesh,
        out_type=jax.ShapeDtypeStruct((OUT_ROWS, D), jnp.float32),
        scratch_types=[
            pltpu.VMEM((b_per_w,), jnp.int32),
            pltpu.VMEM((NCH, CH, D), jnp.float32),
            [pltpu.SemaphoreType.DMA] * NCH,
            pltpu.SemaphoreType.DMA,
        ],
    )
    def gather(table_hbm, idx_hbm, out_hbm, idx_v, rows_v, gsems, wsem):
        wid = lax.axis_index("s") * NC + lax.axis_index("c")
        base = wid * b_per_w
        pltpu.sync_copy(idx_hbm.at[pl.ds(base, b_per_w)], idx_v)
        hs = [pltpu.async_copy(table_hbm.at[idx_v.at[pl.ds(c * CH, CH)]],
                               rows_v.at[c], gsems[c])
              for c in range(NCH)]
        ws = []
        for c in range(NCH):
            hs[c].wait()
            ws.append(pltpu.async_copy(
                rows_v.at[c], out_hbm.at[pl.ds(base + c * CH, CH)], wsem))
        for w in ws:
            w.wait()

    return gather


def kernel(x, W):
    B, T, C = x.shape
    K = W.shape[0]
    H = B // 2
    Wt = W.T
    idx_a, loss_a = _argmin_and_loss(x, Wt, 0, H)
    quant_a = _make_sc_gather(K, C, H * T, B * T)(W, idx_a)
    idx_b, loss_b = _argmin_and_loss(x, Wt, H, B - H)
    quant_b = _make_sc_gather(K, C, H * T, H * T)(W, idx_b)
    quant = lax.dynamic_update_slice(quant_a, quant_b, (H * T, 0))
    quant = quant.reshape(B, T, C)
    idx = jnp.concatenate([idx_a, idx_b]).reshape(B, T)
    loss = jnp.concatenate([loss_a, loss_b])
    return quant, loss, loss, idx


# speedup vs baseline: 1.2353x; 1.0163x over previous
import functools

import jax
import jax.numpy as jnp
from jax import lax
from jax.experimental import pallas as pl
from jax.experimental.pallas import tpu as pltpu
from jax.experimental.pallas import tpu_sc as plsc

_EPS = 1e-5
_BPP = 2


def _argmin_loss_body(x_ref, w_ref, idx_ref, loss_ref):
    nb, T, C = x_ref.shape
    x = x_ref[...].reshape(nb * T, C)
    w = w_ref[...]
    K = w.shape[0]
    R = nb * T

    s = lax.dot_general(x, w, (((1,), (1,)), ((), ())),
                        preferred_element_type=jnp.float32)
    xs = jnp.sum(x * x, axis=1, keepdims=True)
    w2 = jnp.sum(w * w, axis=1)[None, :]
    d2 = (xs + w2) - 2.0 * s

    dmin = jnp.min(d2, axis=1, keepdims=True)
    eqm = d2 == dmin
    kiota = lax.broadcasted_iota(jnp.int32, (R, K), 1).astype(jnp.float32)
    idx = jnp.min(jnp.where(eqm, kiota, float(K)), axis=1)
    p = pl.program_id(0)
    idx_ref[pl.ds(p * R, R)] = idx.astype(jnp.int32)

    ef = eqm.astype(jnp.float32)
    w2_win = lax.dot_general(ef, w2, (((1,), (1,)), ((), ())),
                             preferred_element_type=jnp.float32)
    s_win = 0.5 * ((xs + w2_win) - dmin)
    xn = jnp.maximum(jnp.sqrt(xs), _EPS)
    wn = jnp.maximum(jnp.sqrt(w2_win), _EPS)
    row = (w2_win / (wn * wn) + xs / (xn * xn)
           - 2.0 * s_win / (wn * xn))
    row2 = row.reshape(nb, T)
    for j in range(nb):
        loss_ref[p * nb + j] = jnp.sum(row2[j]) / (T * C)


def _argmin_and_loss(x, W, off, nb):
    B, T, C = x.shape
    K = W.shape[0]
    grid = nb // _BPP
    boff = off // _BPP
    assert off % _BPP == 0
    idxf, loss = pl.pallas_call(
        _argmin_loss_body,
        grid=(grid,),
        in_specs=[
            pl.BlockSpec((_BPP, T, C), lambda b, boff=boff: (b + boff, 0, 0)),
            pl.BlockSpec((K, C), lambda b: (0, 0)),
        ],
        out_specs=[
            pl.BlockSpec((nb * T,), lambda b: (0,)),
            pl.BlockSpec(memory_space=pltpu.SMEM),
        ],
        out_shape=[
            jax.ShapeDtypeStruct((nb * T,), jnp.int32),
            jax.ShapeDtypeStruct((nb,), jnp.float32),
        ],
    )(x, W)
    return idxf, loss


@functools.cache
def _make_sc_gather(V, D, B, OUT_ROWS):
    info = plsc.get_sparse_core_info()
    NC, NS = info.num_cores, info.num_subcores
    NW = NC * NS
    assert B % (8 * NW) == 0
    b_per_w = B // NW
    NCH = 3
    CH = b_per_w // NCH
    assert CH % 8 == 0
    mesh = plsc.VectorSubcoreMesh(core_axis_name="c", subcore_axis_name="s")

    @functools.partial(
        pl.kernel, mesh=mesh,
        out_type=jax.ShapeDtypeStruct((OUT_ROWS, D), jnp.float32),
        scratch_types=[
            pltpu.VMEM((b_per_w,), jnp.int32),
            pltpu.VMEM((NCH, CH, D), jnp.float32),
            [pltpu.SemaphoreType.DMA] * NCH,
            pltpu.SemaphoreType.DMA,
        ],
    )
    def gather(table_hbm, idx_hbm, out_hbm, idx_v, rows_v, gsems, wsem):
        wid = lax.axis_index("s") * NC + lax.axis_index("c")
        base = wid * b_per_w
        pltpu.sync_copy(idx_hbm.at[pl.ds(base, b_per_w)], idx_v)
        hs = [pltpu.async_copy(table_hbm.at[idx_v.at[pl.ds(c * CH, CH)]],
                               rows_v.at[c], gsems[c])
              for c in range(NCH)]
        ws = []
        for c in range(NCH):
            hs[c].wait()
            ws.append(pltpu.async_copy(
                rows_v.at[c], out_hbm.at[pl.ds(base + c * CH, CH)], wsem))
        for w in ws:
            w.wait()

    return gather


def kernel(x, W):
    B, T, C = x.shape
    K = W.shape[0]
    H = B // 2
    idx_a, loss_a = _argmin_and_loss(x, W, 0, H)
    quant_a = _make_sc_gather(K, C, H * T, B * T)(W, idx_a)
    idx_b, loss_b = _argmin_and_loss(x, W, H, B - H)
    quant_b = _make_sc_gather(K, C, H * T, H * T)(W, idx_b)
    quant = lax.dynamic_update_slice(quant_a, quant_b, (H * T, 0))
    quant = quant.reshape(B, T, C)
    idx = jnp.concatenate([idx_a, idx_b]).reshape(B, T)
    loss = jnp.concatenate([loss_a, loss_b])
    return quant, loss, loss, idx
